# Initial kernel scaffold; baseline (speedup 1.0000x reference)
import jax
import jax.numpy as jnp
from jax.experimental import pallas as pl


def kernel(x, edge_list, W1, b1, W2, b2, Wm, bm, Ws, bs):
    raise NotImplementedError("write your pallas kernel here")


# speedup vs baseline: 2.5491x; 2.5491x over previous
import functools

import jax
import jax.numpy as jnp
from jax import lax
from jax.experimental import pallas as pl
from jax.experimental.pallas import tpu as pltpu
from jax.experimental.pallas import tpu_sc as plsc

N = 10000
E = 160000
D = 256
H = 256
NLAYERS = 10
Z = 64

NC = 2
NS = 16
HALF = 128
CHUNK = 128
EPAD = 163840
CPS = EPAD // (NS * CHUNK)
NP = 10240
ROWS_PS = NP // NS


def _sc_agg(h2, srcw, dstw):
    mesh = plsc.VectorSubcoreMesh(
        core_axis_name="c", subcore_axis_name="s", num_cores=NC, num_subcores=NS
    )

    @functools.partial(
        pl.kernel,
        out_type=jax.ShapeDtypeStruct((2 * NP, HALF), jnp.float32),
        mesh=mesh,
        scratch_types=[
            pltpu.VMEM((CPS, CHUNK), jnp.int32),
            pltpu.VMEM((CPS, CHUNK), jnp.int32),
            pltpu.VMEM((CHUNK, HALF), jnp.float32),
            pltpu.VMEM_SHARED((NP, HALF), jnp.float32),
            pltpu.SemaphoreType.DMA,
        ],
    )
    def k(h2_hbm, src_hbm, dst_hbm, out_hbm, sidx, didx, rows, aggsh, gsem):
        c = lax.axis_index("c")
        s = lax.axis_index("s")
        pltpu.sync_copy(src_hbm.at[c, s], sidx)
        pltpu.sync_copy(dst_hbm.at[s], didx)
        pltpu.sync_copy(
            h2_hbm.at[pl.ds(c * NP + s * ROWS_PS, ROWS_PS)],
            aggsh.at[pl.ds(s * ROWS_PS, ROWS_PS)],
        )
        plsc.subcore_barrier()

        def chunk(j, carry):
            pltpu.async_copy(h2_hbm.at[sidx.at[j]], rows, gsem).wait()
            pltpu.sync_copy(rows, aggsh.at[didx.at[j]], add=True)
            return carry

        lax.fori_loop(0, CPS, chunk, 0)
        plsc.subcore_barrier()
        pltpu.sync_copy(
            aggsh.at[pl.ds(s * ROWS_PS, ROWS_PS)],
            out_hbm.at[pl.ds(c * NP + s * ROWS_PS, ROWS_PS)],
        )

    return k(h2, srcw, dstw)


def _mlp(m2, w1, b1, w2, b2, relu):
    R = 1280
    G = NP // R

    def body(m_ref, w1_ref, b1_ref, w2_ref, b2_ref, out_ref):
        m = jnp.concatenate([m_ref[0], m_ref[1]], axis=1)
        t = jnp.dot(m, w1_ref[...], preferred_element_type=jnp.float32) + b1_ref[...]
        t = jnp.maximum(t, 0.0)
        h = jnp.dot(t, w2_ref[...], preferred_element_type=jnp.float32) + b2_ref[...]
        if relu:
            h = jnp.maximum(h, 0.0)
        out_ref[0] = h[:, :HALF]
        out_ref[1] = h[:, HALF:]

    return pl.pallas_call(
        body,
        grid=(G,),
        in_specs=[
            pl.BlockSpec((2, R, HALF), lambda i: (0, i, 0)),
            pl.BlockSpec((H, H), lambda i: (0, 0)),
            pl.BlockSpec((1, H), lambda i: (0, 0)),
            pl.BlockSpec((H, H), lambda i: (0, 0)),
            pl.BlockSpec((1, H), lambda i: (0, 0)),
        ],
        out_specs=pl.BlockSpec((2, R, HALF), lambda i: (0, i, 0)),
        out_shape=jax.ShapeDtypeStruct((2, NP, HALF), jnp.float32),
    )(m2, w1, b1, w2, b2)


def _heads(h2, wcat, bcat):
    R = 1280
    G = NP // R

    def body(h_ref, w_ref, b_ref, out_ref):
        h = jnp.concatenate([h_ref[0], h_ref[1]], axis=1)
        y = jnp.dot(h, w_ref[...], preferred_element_type=jnp.float32) + b_ref[...]
        mean = y[:, :Z]
        x = y[:, Z:]
        sp = jnp.maximum(x, 0.0) + jnp.log(1.0 + jnp.exp(-jnp.abs(x)))
        out_ref[...] = jnp.concatenate([mean, sp], axis=1)

    return pl.pallas_call(
        body,
        grid=(G,),
        in_specs=[
            pl.BlockSpec((2, R, HALF), lambda i: (0, i, 0)),
            pl.BlockSpec((H, 2 * Z), lambda i: (0, 0)),
            pl.BlockSpec((1, 2 * Z), lambda i: (0, 0)),
        ],
        out_specs=pl.BlockSpec((R, 2 * Z), lambda i: (i, 0)),
        out_shape=jax.ShapeDtypeStruct((NP, 2 * Z), jnp.float32),
    )(h2, wcat, bcat)


def kernel(x, edge_list, W1, b1, W2, b2, Wm, bm, Ws, bs):
    src = edge_list[0].astype(jnp.int32)
    dst = edge_list[1].astype(jnp.int32)
    pad = EPAD - E
    src_p = jnp.concatenate([src, jnp.zeros((pad,), jnp.int32)]).reshape(NS, CPS, CHUNK)
    dst_p = jnp.concatenate([dst, jnp.full((pad,), N, jnp.int32)]).reshape(NS, CPS, CHUNK)
    srcw = jnp.stack([src_p, src_p + NP])
    zpad = jnp.zeros((NP - N, HALF), jnp.float32)
    h2 = jnp.concatenate([x[:, :HALF], zpad, x[:, HALF:], zpad], axis=0)

    b1r = b1.reshape(NLAYERS, 1, H)
    b2r = b2.reshape(NLAYERS, 1, H)
    for i in range(NLAYERS):
        m2 = _sc_agg(h2, srcw, dst_p)
        h2 = _mlp(
            m2.reshape(2, NP, HALF), W1[i], b1r[i], W2[i], b2r[i],
            relu=(i < NLAYERS - 1),
        ).reshape(2 * NP, HALF)

    wcat = jnp.concatenate([Wm, Ws], axis=1)
    bcat = jnp.concatenate([bm, bs]).reshape(1, 2 * Z)
    y = _heads(h2.reshape(2, NP, HALF), wcat, bcat)
    return y[:N, :Z], y[:N, Z:]


# speedup vs baseline: 2.5889x; 1.0156x over previous
import functools

import jax
import jax.numpy as jnp
from jax import lax
from jax.experimental import pallas as pl
from jax.experimental.pallas import tpu as pltpu
from jax.experimental.pallas import tpu_sc as plsc

N = 10000
E = 160000
D = 256
H = 256
NLAYERS = 10
Z = 64

NC = 2
NS = 16
HALF = 128
CHUNK = 128
EPAD = 163840
CPS = EPAD // (NS * CHUNK)
NP = 10240
GW = 4
CPG = CPS // GW
ROWS_PS = NP // NS


def _sc_agg(h2, srcw, dstw):
    mesh = plsc.VectorSubcoreMesh(
        core_axis_name="c", subcore_axis_name="s", num_cores=NC, num_subcores=NS
    )

    @functools.partial(
        pl.kernel,
        out_type=jax.ShapeDtypeStruct((2 * NP, HALF), jnp.float32),
        mesh=mesh,
        scratch_types=[
            pltpu.VMEM((CPS, CHUNK), jnp.int32),
            pltpu.VMEM((CPS, CHUNK), jnp.int32),
            pltpu.VMEM((CHUNK, HALF), jnp.float32),
            pltpu.SemaphoreType.DMA,
            pltpu.VMEM_SHARED((NP, HALF), jnp.float32),
        ],
    )
    def k(h2_hbm, src_hbm, dst_hbm, out_hbm, sidx, didx, rows, gsem, aggsh):
        c = lax.axis_index("c")
        s = lax.axis_index("s")
        pltpu.sync_copy(src_hbm.at[c, s], sidx)
        pltpu.sync_copy(dst_hbm.at[s], didx)
        pltpu.sync_copy(
            h2_hbm.at[pl.ds(c * NP + s * ROWS_PS, ROWS_PS)],
            aggsh.at[pl.ds(s * ROWS_PS, ROWS_PS)],
        )
        plsc.subcore_barrier()

        def chunk(j, carry):
            pltpu.async_copy(h2_hbm.at[sidx.at[j]], rows, gsem).wait()
            pltpu.sync_copy(rows, aggsh.at[didx.at[j]], add=True)
            return carry

        lax.fori_loop(0, CPS, chunk, 0)
        plsc.subcore_barrier()
        pltpu.sync_copy(
            aggsh.at[pl.ds(s * ROWS_PS, ROWS_PS)],
            out_hbm.at[pl.ds(c * NP + s * ROWS_PS, ROWS_PS)],
        )

    return k(h2, srcw, dstw)


def _mlp(m2, w1, b1, w2, b2, relu):
    R = 1280
    G = NP // R

    def body(m_ref, w1_ref, b1_ref, w2_ref, b2_ref, out_ref):
        m = jnp.concatenate([m_ref[0], m_ref[1]], axis=1)
        t = jnp.dot(m, w1_ref[...], preferred_element_type=jnp.float32) + b1_ref[...]
        t = jnp.maximum(t, 0.0)
        h = jnp.dot(t, w2_ref[...], preferred_element_type=jnp.float32) + b2_ref[...]
        if relu:
            h = jnp.maximum(h, 0.0)
        out_ref[0] = h[:, :HALF]
        out_ref[1] = h[:, HALF:]

    return pl.pallas_call(
        body,
        grid=(G,),
        in_specs=[
            pl.BlockSpec((2, R, HALF), lambda i: (0, i, 0)),
            pl.BlockSpec((H, H), lambda i: (0, 0)),
            pl.BlockSpec((1, H), lambda i: (0, 0)),
            pl.BlockSpec((H, H), lambda i: (0, 0)),
            pl.BlockSpec((1, H), lambda i: (0, 0)),
        ],
        out_specs=pl.BlockSpec((2, R, HALF), lambda i: (0, i, 0)),
        out_shape=jax.ShapeDtypeStruct((2, NP, HALF), jnp.float32),
    )(m2, w1, b1, w2, b2)


def _heads(h2, wcat, bcat):
    R = 1280
    G = NP // R

    def body(h_ref, w_ref, b_ref, out_ref):
        h = jnp.concatenate([h_ref[0], h_ref[1]], axis=1)
        y = jnp.dot(h, w_ref[...], preferred_element_type=jnp.float32) + b_ref[...]
        mean = y[:, :Z]
        x = y[:, Z:]
        sp = jnp.maximum(x, 0.0) + jnp.log(1.0 + jnp.exp(-jnp.abs(x)))
        out_ref[...] = jnp.concatenate([mean, sp], axis=1)

    return pl.pallas_call(
        body,
        grid=(G,),
        in_specs=[
            pl.BlockSpec((2, R, HALF), lambda i: (0, i, 0)),
            pl.BlockSpec((H, 2 * Z), lambda i: (0, 0)),
            pl.BlockSpec((1, 2 * Z), lambda i: (0, 0)),
        ],
        out_specs=pl.BlockSpec((R, 2 * Z), lambda i: (i, 0)),
        out_shape=jax.ShapeDtypeStruct((NP, 2 * Z), jnp.float32),
    )(h2, wcat, bcat)


def kernel(x, edge_list, W1, b1, W2, b2, Wm, bm, Ws, bs):
    src = edge_list[0].astype(jnp.int32)
    dst = edge_list[1].astype(jnp.int32)
    order = jnp.argsort(dst)
    src = src[order]
    dst = dst[order]
    pad = EPAD - E
    src_p = jnp.concatenate([src, jnp.zeros((pad,), jnp.int32)]).reshape(NS, CPS, CHUNK)
    dst_p = jnp.concatenate([dst, jnp.full((pad,), N, jnp.int32)]).reshape(NS, CPS, CHUNK)
    srcw = jnp.stack([src_p, src_p + NP])
    zpad = jnp.zeros((NP - N, HALF), jnp.float32)
    h2 = jnp.concatenate([x[:, :HALF], zpad, x[:, HALF:], zpad], axis=0)

    b1r = b1.reshape(NLAYERS, 1, H)
    b2r = b2.reshape(NLAYERS, 1, H)
    for i in range(NLAYERS):
        m2 = _sc_agg(h2, srcw, dst_p)
        h2 = _mlp(
            m2.reshape(2, NP, HALF), W1[i], b1r[i], W2[i], b2r[i],
            relu=(i < NLAYERS - 1),
        ).reshape(2 * NP, HALF)

    wcat = jnp.concatenate([Wm, Ws], axis=1)
    bcat = jnp.concatenate([bm, bs]).reshape(1, 2 * Z)
    y = _heads(h2.reshape(2, NP, HALF), wcat, bcat)
    return y[:N, :Z], y[:N, Z:]
